# baseline (speedup 1.0000x reference)
import functools

import jax
import jax.numpy as jnp
from jax import lax
from jax.experimental import pallas as pl
from jax.experimental.pallas import tpu as pltpu


def _bf16(w):
    return w.astype(jnp.bfloat16)


def _mlp_max_kernel(x_ref, *refs, per_batch, last_relu, n_layers):
    *prefs, o_ref = refs
    h = x_ref[0].astype(jnp.bfloat16)
    wl_ref = prefs[2 * (n_layers - 1)]
    wl = wl_ref[0] if per_batch[n_layers - 1] else wl_ref[...]
    c_last = wl.shape[-1]

    for i in range(n_layers - 1):
        w_ref, b_ref = prefs[2 * i], prefs[2 * i + 1]
        w = w_ref[0] if per_batch[i] else w_ref[...]
        h = lax.dot_general(w, h, (((0,), (0,)), ((), ())),
                            preferred_element_type=jnp.float32)
        h = jnp.maximum(h + b_ref[...], 0.0).astype(jnp.bfloat16)

    part = jnp.full((c_last, 128), -jnp.inf, jnp.float32)
    for c in range(0, h.shape[-1], 256):
        y = lax.dot_general(wl, h[:, c:c + 256], (((0,), (0,)), ((), ())),
                            preferred_element_type=jnp.float32)
        part = jnp.maximum(part, jnp.maximum(y[:, :128], y[:, 128:]))

    a = part.T
    a = jnp.max(a.reshape(16, 8, a.shape[-1]), axis=0)
    r = jnp.max(a, axis=0, keepdims=True) + prefs[-1][...]
    if last_relu:
        r = jnp.maximum(r, 0.0)
    o_ref[0] = r


def _mlp_maxpool(x_bcn, layers, last_relu):
    B, _, N = x_bcn.shape
    c_last = layers[-1][0].shape[-1]
    per_batch = tuple(w.ndim == 3 for (w, _) in layers)

    in_specs = [pl.BlockSpec((1, 3, N), lambda b: (b, 0, 0))]
    args = [x_bcn]
    for wb, sh in layers:
        if wb.ndim == 3:
            in_specs.append(pl.BlockSpec((1,) + wb.shape[1:],
                                         lambda b: (b, 0, 0)))
        else:
            in_specs.append(pl.BlockSpec(wb.shape, lambda b: (0, 0)))
        in_specs.append(pl.BlockSpec(sh.shape, lambda b: (0, 0)))
        args += [wb, sh]

    out = pl.pallas_call(
        functools.partial(_mlp_max_kernel, per_batch=per_batch,
                          last_relu=last_relu, n_layers=len(layers)),
        out_shape=jax.ShapeDtypeStruct((B, 1, c_last), jnp.float32),
        grid=(B,),
        in_specs=in_specs,
        out_specs=pl.BlockSpec((1, 1, c_last), lambda b: (b, 0, 0)),
        compiler_params=pltpu.CompilerParams(
            dimension_semantics=("parallel",)),
    )(*args)
    return out.reshape(B, c_last)


def _head_kernel(x_ref, w1, s1, b1, w2, s2, b2, w3, s3, b3, o_ref, *, logsm):
    h = jnp.dot(x_ref[...].astype(jnp.bfloat16), w1[...],
                preferred_element_type=jnp.float32)
    h = jnp.maximum(h * s1[...] + b1[...], 0.0)
    h = jnp.dot(h.astype(jnp.bfloat16), w2[...],
                preferred_element_type=jnp.float32)
    h = jnp.maximum(h * s2[...] + b2[...], 0.0)
    z = jnp.dot(h.astype(jnp.bfloat16), w3[...],
                preferred_element_type=jnp.float32)
    z = z * s3[...] + b3[...]
    if logsm:
        z = z - jnp.max(z, axis=-1, keepdims=True)
        z = z - jnp.log(jnp.sum(jnp.exp(z), axis=-1, keepdims=True))
    o_ref[...] = z


def _head(pooled, l1, l2, l3, *, logsm):
    B = pooled.shape[0]
    K = l3[0].shape[-1]
    full = lambda shape: pl.BlockSpec(shape, lambda: (0,) * len(shape))
    args, specs = [pooled], [full(pooled.shape)]
    for w, s, sh in (l1, l2, l3):
        args += [w, s, sh]
        specs += [full(w.shape), full(s.shape), full(sh.shape)]
    return pl.pallas_call(
        functools.partial(_head_kernel, logsm=logsm),
        out_shape=jax.ShapeDtypeStruct((B, K), jnp.float32),
        in_specs=specs,
        out_specs=full((B, K)),
    )(*args)


def _col(sh):
    return sh.reshape(-1, 1)


def _forward(x, p):
    B = x.shape[0]

    pooled = _mlp_maxpool(x, p["t3_convs"], last_relu=True)
    z3 = _head(pooled, *p["t3_head"], logsm=False)
    m3 = z3.reshape(B, 3, 3) + jnp.eye(3, dtype=jnp.float32)[None]

    w1f = _bf16(jnp.matmul(m3, p["conv1_w"]) * p["conv1_scale"])
    conv1f = (w1f, p["conv1_shift"])

    pooled = _mlp_maxpool(x, [conv1f] + p["t64_convs"], last_relu=True)
    z64 = _head(pooled, *p["t64_head"], logsm=False)
    m64 = z64.reshape(B, 64, 64) + jnp.eye(64, dtype=jnp.float32)[None]

    w2f = _bf16(jnp.matmul(m64, p["conv2_w"]) * p["conv2_scale"])
    conv2f = (w2f, p["conv2_shift"])

    feat = _mlp_maxpool(x, [conv1f, conv2f, p["conv3"]], last_relu=False)
    out = _head(feat, *p["final_head"], logsm=True)
    return out, m3, m64


def kernel(x, transform_tnet3_conv1_w, transform_tnet3_conv1_scale, transform_tnet3_conv1_shift, transform_tnet3_conv2_w, transform_tnet3_conv2_scale, transform_tnet3_conv2_shift, transform_tnet3_conv3_w, transform_tnet3_conv3_scale, transform_tnet3_conv3_shift, transform_tnet3_fc1_w, transform_tnet3_fc1_scale, transform_tnet3_fc1_shift, transform_tnet3_fc2_w, transform_tnet3_fc2_scale, transform_tnet3_fc2_shift, transform_tnet3_fc3_w, transform_tnet3_fc3_scale, transform_tnet3_fc3_shift, transform_tnet64_conv1_w, transform_tnet64_conv1_scale, transform_tnet64_conv1_shift, transform_tnet64_conv2_w, transform_tnet64_conv2_scale, transform_tnet64_conv2_shift, transform_tnet64_conv3_w, transform_tnet64_conv3_scale, transform_tnet64_conv3_shift, transform_tnet64_fc1_w, transform_tnet64_fc1_scale, transform_tnet64_fc1_shift, transform_tnet64_fc2_w, transform_tnet64_fc2_scale, transform_tnet64_fc2_shift, transform_tnet64_fc3_w, transform_tnet64_fc3_scale, transform_tnet64_fc3_shift, transform_conv1_w, transform_conv1_scale, transform_conv1_shift, transform_conv2_w, transform_conv2_scale, transform_conv2_shift, transform_conv3_w, transform_conv3_scale, transform_conv3_shift, fc1_w, fc1_scale, fc1_shift, fc2_w, fc2_scale, fc2_shift, fc3_w, fc3_scale, fc3_shift):
    conv = lambda w, s, sh: (_bf16(w * s), _col(sh))
    last = lambda w, s, sh: (_bf16(w * s), sh)
    head = lambda w, s, sh: (_bf16(w), s, sh)
    p = {
        "t3_convs": [
            conv(transform_tnet3_conv1_w, transform_tnet3_conv1_scale, transform_tnet3_conv1_shift),
            conv(transform_tnet3_conv2_w, transform_tnet3_conv2_scale, transform_tnet3_conv2_shift),
            last(transform_tnet3_conv3_w, transform_tnet3_conv3_scale, transform_tnet3_conv3_shift),
        ],
        "t3_head": (
            head(transform_tnet3_fc1_w, transform_tnet3_fc1_scale, transform_tnet3_fc1_shift),
            head(transform_tnet3_fc2_w, transform_tnet3_fc2_scale, transform_tnet3_fc2_shift),
            head(transform_tnet3_fc3_w, transform_tnet3_fc3_scale, transform_tnet3_fc3_shift),
        ),
        "t64_convs": [
            conv(transform_tnet64_conv1_w, transform_tnet64_conv1_scale, transform_tnet64_conv1_shift),
            conv(transform_tnet64_conv2_w, transform_tnet64_conv2_scale, transform_tnet64_conv2_shift),
            last(transform_tnet64_conv3_w, transform_tnet64_conv3_scale, transform_tnet64_conv3_shift),
        ],
        "t64_head": (
            head(transform_tnet64_fc1_w, transform_tnet64_fc1_scale, transform_tnet64_fc1_shift),
            head(transform_tnet64_fc2_w, transform_tnet64_fc2_scale, transform_tnet64_fc2_shift),
            head(transform_tnet64_fc3_w, transform_tnet64_fc3_scale, transform_tnet64_fc3_shift),
        ),
        "conv1_w": transform_conv1_w,
        "conv1_scale": transform_conv1_scale,
        "conv1_shift": _col(transform_conv1_shift),
        "conv2_w": transform_conv2_w,
        "conv2_scale": transform_conv2_scale,
        "conv2_shift": _col(transform_conv2_shift),
        "conv3": last(transform_conv3_w, transform_conv3_scale, transform_conv3_shift),
        "final_head": (
            head(fc1_w, fc1_scale, fc1_shift),
            head(fc2_w, fc2_scale, fc2_shift),
            head(fc3_w, fc3_scale, fc3_shift),
        ),
    }

    devs = jax.devices()
    nd = 2 if len(devs) >= 2 and x.shape[0] % 2 == 0 else 1
    if nd == 1:
        return _forward(x, p)
    mesh = jax.sharding.Mesh(devs[:nd], ("d",))
    P = jax.sharding.PartitionSpec
    fwd = jax.shard_map(
        _forward, mesh=mesh,
        in_specs=(P("d"), jax.tree.map(lambda _: P(), p)),
        out_specs=(P("d"), P("d"), P("d")), check_vma=False)
    return fwd(x, p)


# speedup vs baseline: 3.0330x; 3.0330x over previous
import functools

import jax
import jax.numpy as jnp
from jax import lax
from jax.experimental import pallas as pl
from jax.experimental.pallas import tpu as pltpu


def _bf16(w):
    return w.astype(jnp.bfloat16)


def _mlp_max_kernel(x_ref, *refs, per_batch, last_relu, n_layers):
    *prefs, o_ref = refs
    h = x_ref[0].astype(jnp.bfloat16)
    wl_ref = prefs[2 * (n_layers - 1)]
    wl = wl_ref[0] if per_batch[n_layers - 1] else wl_ref[...]
    c_last = wl.shape[-1]

    for i in range(n_layers - 1):
        w_ref, b_ref = prefs[2 * i], prefs[2 * i + 1]
        w = w_ref[0] if per_batch[i] else w_ref[...]
        h = lax.dot_general(w, h, (((0,), (0,)), ((), ())),
                            preferred_element_type=jnp.float32)
        h = jnp.maximum(h + b_ref[...], 0.0).astype(jnp.bfloat16)

    part = jnp.full((c_last, 128), -jnp.inf, jnp.float32)
    for c in range(0, h.shape[-1], 256):
        y = lax.dot_general(wl, h[:, c:c + 256], (((0,), (0,)), ((), ())),
                            preferred_element_type=jnp.float32)
        part = jnp.maximum(part, jnp.maximum(y[:, :128], y[:, 128:]))

    a = part.T
    a = jnp.max(a.reshape(16, 8, a.shape[-1]), axis=0)
    r = jnp.max(a, axis=0, keepdims=True) + prefs[-1][...]
    if last_relu:
        r = jnp.maximum(r, 0.0)
    o_ref[0] = r


def _mlp_maxpool(x_bcn, layers, last_relu):
    B, _, N = x_bcn.shape
    c_last = layers[-1][0].shape[-1]
    per_batch = tuple(w.ndim == 3 for (w, _) in layers)

    in_specs = [pl.BlockSpec((1, 3, N), lambda b: (b, 0, 0))]
    args = [x_bcn]
    for wb, sh in layers:
        if wb.ndim == 3:
            in_specs.append(pl.BlockSpec((1,) + wb.shape[1:],
                                         lambda b: (b, 0, 0)))
        else:
            in_specs.append(pl.BlockSpec(wb.shape, lambda b: (0, 0)))
        in_specs.append(pl.BlockSpec(sh.shape, lambda b: (0, 0)))
        args += [wb, sh]

    out = pl.pallas_call(
        functools.partial(_mlp_max_kernel, per_batch=per_batch,
                          last_relu=last_relu, n_layers=len(layers)),
        out_shape=jax.ShapeDtypeStruct((B, 1, c_last), jnp.float32),
        grid=(B,),
        in_specs=in_specs,
        out_specs=pl.BlockSpec((1, 1, c_last), lambda b: (b, 0, 0)),
        compiler_params=pltpu.CompilerParams(
            dimension_semantics=("parallel",)),
    )(*args)
    return out.reshape(B, c_last)


def _head_kernel(x_ref, w1, s1, b1, w2, s2, b2, w3, s3, b3, o_ref, *, logsm):
    h = jnp.dot(x_ref[...].astype(jnp.bfloat16), w1[...],
                preferred_element_type=jnp.float32)
    h = jnp.maximum(h * s1[...] + b1[...], 0.0)
    h = jnp.dot(h.astype(jnp.bfloat16), w2[...],
                preferred_element_type=jnp.float32)
    h = jnp.maximum(h * s2[...] + b2[...], 0.0)
    z = jnp.dot(h.astype(jnp.bfloat16), w3[...],
                preferred_element_type=jnp.float32)
    z = z * s3[...] + b3[...]
    if logsm:
        z = z - jnp.max(z, axis=-1, keepdims=True)
        z = z - jnp.log(jnp.sum(jnp.exp(z), axis=-1, keepdims=True))
    o_ref[...] = z


def _head(pooled, l1, l2, l3, *, logsm):
    B = pooled.shape[0]
    K = l3[0].shape[-1]
    full = lambda shape: pl.BlockSpec(shape, lambda: (0,) * len(shape))
    args, specs = [pooled], [full(pooled.shape)]
    for w, s, sh in (l1, l2, l3):
        args += [w, s, sh]
        specs += [full(w.shape), full(s.shape), full(sh.shape)]
    return pl.pallas_call(
        functools.partial(_head_kernel, logsm=logsm),
        out_shape=jax.ShapeDtypeStruct((B, K), jnp.float32),
        in_specs=specs,
        out_specs=full((B, K)),
    )(*args)


def _col(sh):
    return sh.reshape(-1, 1)


def _forward(x, *flat):
    p = _prep(*flat)
    B = x.shape[0]

    pooled = _mlp_maxpool(x, p["t3_convs"], last_relu=True)
    z3 = _head(pooled, *p["t3_head"], logsm=False)
    m3 = z3.reshape(B, 3, 3) + jnp.eye(3, dtype=jnp.float32)[None]

    w1f = _bf16(jnp.matmul(m3, p["conv1_w"]) * p["conv1_scale"])
    conv1f = (w1f, p["conv1_shift"])

    pooled = _mlp_maxpool(x, [conv1f] + p["t64_convs"], last_relu=True)
    z64 = _head(pooled, *p["t64_head"], logsm=False)
    m64 = z64.reshape(B, 64, 64) + jnp.eye(64, dtype=jnp.float32)[None]

    w2f = _bf16(jnp.matmul(m64, p["conv2_w"]) * p["conv2_scale"])
    conv2f = (w2f, p["conv2_shift"])

    feat = _mlp_maxpool(x, [conv1f, conv2f, p["conv3"]], last_relu=False)
    out = _head(feat, *p["final_head"], logsm=True)
    return out, m3, m64


def _prep(transform_tnet3_conv1_w, transform_tnet3_conv1_scale, transform_tnet3_conv1_shift, transform_tnet3_conv2_w, transform_tnet3_conv2_scale, transform_tnet3_conv2_shift, transform_tnet3_conv3_w, transform_tnet3_conv3_scale, transform_tnet3_conv3_shift, transform_tnet3_fc1_w, transform_tnet3_fc1_scale, transform_tnet3_fc1_shift, transform_tnet3_fc2_w, transform_tnet3_fc2_scale, transform_tnet3_fc2_shift, transform_tnet3_fc3_w, transform_tnet3_fc3_scale, transform_tnet3_fc3_shift, transform_tnet64_conv1_w, transform_tnet64_conv1_scale, transform_tnet64_conv1_shift, transform_tnet64_conv2_w, transform_tnet64_conv2_scale, transform_tnet64_conv2_shift, transform_tnet64_conv3_w, transform_tnet64_conv3_scale, transform_tnet64_conv3_shift, transform_tnet64_fc1_w, transform_tnet64_fc1_scale, transform_tnet64_fc1_shift, transform_tnet64_fc2_w, transform_tnet64_fc2_scale, transform_tnet64_fc2_shift, transform_tnet64_fc3_w, transform_tnet64_fc3_scale, transform_tnet64_fc3_shift, transform_conv1_w, transform_conv1_scale, transform_conv1_shift, transform_conv2_w, transform_conv2_scale, transform_conv2_shift, transform_conv3_w, transform_conv3_scale, transform_conv3_shift, fc1_w, fc1_scale, fc1_shift, fc2_w, fc2_scale, fc2_shift, fc3_w, fc3_scale, fc3_shift):
    conv = lambda w, s, sh: (_bf16(w * s), _col(sh))
    last = lambda w, s, sh: (_bf16(w * s), sh)
    head = lambda w, s, sh: (_bf16(w), s, sh)
    p = {
        "t3_convs": [
            conv(transform_tnet3_conv1_w, transform_tnet3_conv1_scale, transform_tnet3_conv1_shift),
            conv(transform_tnet3_conv2_w, transform_tnet3_conv2_scale, transform_tnet3_conv2_shift),
            last(transform_tnet3_conv3_w, transform_tnet3_conv3_scale, transform_tnet3_conv3_shift),
        ],
        "t3_head": (
            head(transform_tnet3_fc1_w, transform_tnet3_fc1_scale, transform_tnet3_fc1_shift),
            head(transform_tnet3_fc2_w, transform_tnet3_fc2_scale, transform_tnet3_fc2_shift),
            head(transform_tnet3_fc3_w, transform_tnet3_fc3_scale, transform_tnet3_fc3_shift),
        ),
        "t64_convs": [
            conv(transform_tnet64_conv1_w, transform_tnet64_conv1_scale, transform_tnet64_conv1_shift),
            conv(transform_tnet64_conv2_w, transform_tnet64_conv2_scale, transform_tnet64_conv2_shift),
            last(transform_tnet64_conv3_w, transform_tnet64_conv3_scale, transform_tnet64_conv3_shift),
        ],
        "t64_head": (
            head(transform_tnet64_fc1_w, transform_tnet64_fc1_scale, transform_tnet64_fc1_shift),
            head(transform_tnet64_fc2_w, transform_tnet64_fc2_scale, transform_tnet64_fc2_shift),
            head(transform_tnet64_fc3_w, transform_tnet64_fc3_scale, transform_tnet64_fc3_shift),
        ),
        "conv1_w": transform_conv1_w,
        "conv1_scale": transform_conv1_scale,
        "conv1_shift": _col(transform_conv1_shift),
        "conv2_w": transform_conv2_w,
        "conv2_scale": transform_conv2_scale,
        "conv2_shift": _col(transform_conv2_shift),
        "conv3": last(transform_conv3_w, transform_conv3_scale, transform_conv3_shift),
        "final_head": (
            head(fc1_w, fc1_scale, fc1_shift),
            head(fc2_w, fc2_scale, fc2_shift),
            head(fc3_w, fc3_scale, fc3_shift),
        ),
    }
    return p


def kernel(*args):
    x = args[0]
    devs = jax.devices()
    nd = 2 if len(devs) >= 2 and x.shape[0] % 2 == 0 else 1
    if nd == 1:
        return _forward(*args)
    mesh = jax.sharding.Mesh(devs[:nd], ("d",))
    P = jax.sharding.PartitionSpec
    fwd = jax.shard_map(
        _forward, mesh=mesh,
        in_specs=(P("d"),) + (P(),) * (len(args) - 1),
        out_specs=(P("d"), P("d"), P("d")), check_vma=False)
    return fwd(*args)
